# Initial kernel scaffold; baseline (speedup 1.0000x reference)
import jax
import jax.numpy as jnp
from jax.experimental import pallas as pl


def kernel(obs, policies, actions, weights, Wk1, bk1, Wk2, bk2, Wq1, bq1, Wq2, bq2, Wv1, bv1, Wv2, bv2, Wval, bval):
    raise NotImplementedError("write your pallas kernel here")


# speedup vs baseline: 3.1652x; 3.1652x over previous
import functools
import math

import jax
import jax.numpy as jnp
from jax.experimental import pallas as pl

_N_AGENTS = 32
_N_ACTIONS = 8
_ROWS = 256


def _body(obs_ref, pol_ref, act_ref, noise_ref,
          wk1_ref, bk1_ref, wk2_ref, bk2_ref,
          wq1_ref, bq1_ref, wq2_ref, bq2_ref,
          wv1_ref, bv1_ref, wv2_ref, bv2_ref,
          wd_ref, wa_ref, wrep_ref, scal_ref,
          x_ref, alpha_ref):
    n = _N_AGENTS
    R = _ROWS
    f32 = jnp.float32

    obs = obs_ref[...]
    h = jnp.tanh(jnp.dot(obs, wk1_ref[...], preferred_element_type=f32) + bk1_ref[...])
    kf = jnp.dot(h, wk2_ref[...], preferred_element_type=f32) + bk2_ref[...]
    h = jnp.tanh(jnp.dot(obs, wq1_ref[...], preferred_element_type=f32) + bq1_ref[...])
    qf = jnp.dot(h, wq2_ref[...], preferred_element_type=f32) + bq2_ref[...]
    h = jnp.tanh(jnp.dot(obs, wv1_ref[...], preferred_element_type=f32) + bv1_ref[...])
    vf = jnp.dot(h, wv2_ref[...], preferred_element_type=f32) + bv2_ref[...]

    ii = jax.lax.broadcasted_iota(jnp.int32, (R, R), 0)
    jj = jax.lax.broadcasted_iota(jnp.int32, (R, R), 1)
    same = (ii // n) == (jj // n)
    tmat = same.astype(f32)

    scores = jnp.dot(qf, kf.T, preferred_element_type=f32) * (1.0 / math.sqrt(32.0))
    scores = jnp.where(same, scores, -1e30)
    m = jnp.max(scores, axis=1, keepdims=True)
    e = jnp.exp(scores - m) * tmat
    alpha = e / jnp.sum(e, axis=1, keepdims=True)

    j2 = jax.lax.broadcasted_iota(jnp.int32, (R, n), 0)
    q2 = jax.lax.broadcasted_iota(jnp.int32, (R, n), 1)
    sel = ((j2 % n) == q2).astype(f32)

    alpha_ref[...] = jnp.dot(alpha, sel, preferred_element_type=f32)

    obs_proc = jnp.dot(alpha, vf, preferred_element_type=f32)
    u = jnp.sum(obs_proc * wd_ref[...], axis=1, keepdims=True)

    w = scal_ref[0, 0]
    bval = scal_ref[0, 1]
    pol = pol_ref[...]
    act = act_ref[...]
    wa = wa_ref[...]
    zb = w * act + (1.0 - w) * pol
    base = jnp.sum(zb * wa, axis=1, keepdims=True)
    polw = jnp.sum(pol * wa, axis=1, keepdims=True)

    inv_n = 1.0 / n
    t = u + bval + (polw - base) * inv_n
    tb = jnp.dot(tmat, sel * t, preferred_element_type=f32)
    base_b = jnp.dot(tmat, sel * base, preferred_element_type=f32)
    sum_base = jnp.sum(base_b, axis=1, keepdims=True)

    nw = jnp.dot(noise_ref[...], wrep_ref[...], preferred_element_type=f32)
    row = (sum_base + jnp.sum(nw, axis=1, keepdims=True)) * inv_n
    x_ref[...] = tb + row - nw * inv_n


@jax.jit
def kernel(obs, policies, actions, weights, Wk1, bk1, Wk2, bk2,
           Wq1, bq1, Wq2, bq2, Wv1, bv1, Wv2, bv2, Wval, bval):
    n = _N_AGENTS
    A = _N_ACTIONS
    Ntot = obs.shape[0]
    d = Wk2.shape[1]
    R = _ROWS
    grid = Ntot // R

    noise = jax.random.normal(jax.random.key(42), (Ntot, n * A), dtype=jnp.float32) * 0.1

    wd = Wval[:d, 0].reshape(1, d)
    wa = Wval[d:, 0].reshape(1, A)
    wrep = jnp.kron(jnp.eye(n, dtype=jnp.float32), Wval[d:, 0].reshape(A, 1))
    scal = jnp.stack([weights[0], bval[0]]).reshape(1, 2)

    row_spec = lambda c: pl.BlockSpec((R, c), lambda g: (g, 0))
    full = lambda arr: pl.BlockSpec(arr.shape, lambda g: (0, 0))

    x, alpha = pl.pallas_call(
        _body,
        grid=(grid,),
        in_specs=[
            row_spec(obs.shape[1]),
            row_spec(A), row_spec(A),
            row_spec(n * A),
            full(Wk1), full(bk1.reshape(1, -1)), full(Wk2), full(bk2.reshape(1, -1)),
            full(Wq1), full(bq1.reshape(1, -1)), full(Wq2), full(bq2.reshape(1, -1)),
            full(Wv1), full(bv1.reshape(1, -1)), full(Wv2), full(bv2.reshape(1, -1)),
            full(wd), full(wa), full(wrep), full(scal),
        ],
        out_specs=[row_spec(n), row_spec(n)],
        out_shape=[
            jax.ShapeDtypeStruct((Ntot, n), jnp.float32),
            jax.ShapeDtypeStruct((Ntot, n), jnp.float32),
        ],
    )(obs, policies, actions, noise,
      Wk1, bk1.reshape(1, -1), Wk2, bk2.reshape(1, -1),
      Wq1, bq1.reshape(1, -1), Wq2, bq2.reshape(1, -1),
      Wv1, bv1.reshape(1, -1), Wv2, bv2.reshape(1, -1),
      wd, wa, wrep, scal)

    return x.reshape(Ntot, n, 1), alpha.reshape(Ntot, n, 1)
